# baseline (speedup 1.0000x reference)
import functools

import ml_dtypes
import numpy as np

import jax
import jax.numpy as jnp
from jax import lax
from jax.experimental import pallas as pl
from jax.experimental.pallas import tpu as pltpu
from jax.experimental.pallas import tpu_sc as plsc

N = 4096
NC = 2
NS = 16
NW = NC * NS
QPW = N // NW
LANES = 16
Q32 = QPW // 32

BIG = np.float32(1e30)
BF16 = ml_dtypes.bfloat16

def _sqrt16(x):
    i = plsc.bitcast(x, jnp.int32)
    i = (i >> 1) + np.int32(0x1FBD1DF5)
    y = plsc.bitcast(i, jnp.float32)
    for _ in range(3):
        y = np.float32(0.5) * (y + x / y)
    return jnp.where(x > 0.0, y, np.float32(0.0))


def _knn_body(pts_t_hbm, out_hbm, pts_v, outv):
    wid = lax.axis_index("s") * NC + lax.axis_index("c")
    base = wid * QPW
    pltpu.sync_copy(pts_t_hbm, pts_v)

    zeros = jnp.zeros((LANES,), jnp.float32)
    fmt = plsc.PackFormat.INTERLEAVED

    qx, qy, qz = [], [], []
    for u in range(Q32):
        lo = pl.ds(base + u * 32, LANES)
        hi = pl.ds(base + u * 32 + LANES, LANES)
        qx.append(plsc.pack(pts_v[0, lo], pts_v[0, hi], format=fmt))
        qy.append(plsc.pack(pts_v[1, lo], pts_v[1, hi], format=fmt))
        qz.append(plsc.pack(pts_v[2, lo], pts_v[2, hi], format=fmt))

    lane_iota = lax.iota(jnp.int32, LANES)

    def step(jv, carry, diag_k=None):
        m1, m2, m3 = (list(c) for c in carry)
        off = pl.multiple_of(jv * LANES, LANES)
        csl = pl.ds(off, LANES)
        cxv = pts_v[0, csl]
        cyv = pts_v[1, csl]
        czv = pts_v[2, csl]
        for l in range(LANES):
            cxs = jnp.broadcast_to(cxv[l], (LANES,))
            cys = jnp.broadcast_to(cyv[l], (LANES,))
            czs = jnp.broadcast_to(czv[l], (LANES,))
            cxb = plsc.pack(cxs, cxs, format=fmt)
            cyb = plsc.pack(cys, cys, format=fmt)
            czb = plsc.pack(czs, czs, format=fmt)
            if diag_k is not None:
                dsel = jnp.where(lane_iota == l, BIG, np.float32(0.0))
                halves = (dsel, zeros) if diag_k % 2 == 0 else (zeros, dsel)
                dvec = plsc.pack(halves[0], halves[1], format=fmt)
            for u in range(Q32):
                dx = qx[u] - cxb
                s = dx * dx
                dy = qy[u] - cyb
                s = s + dy * dy
                dz = qz[u] - czb
                s = s + dz * dz
                if diag_k is not None and u == diag_k // 2:
                    s = s + dvec
                hi1 = jnp.maximum(m1[u], s)
                m1[u] = jnp.minimum(m1[u], s)
                hi2 = jnp.maximum(m2[u], hi1)
                m2[u] = jnp.minimum(m2[u], hi1)
                m3[u] = jnp.minimum(m3[u], hi2)
        return m1, m2, m3

    big16 = jnp.full((32,), 1e30, jnp.bfloat16)
    init = tuple([big16 for _ in range(Q32)] for _ in range(3))
    qblk = base // LANES
    carry = plsc.parallel_loop(0, qblk, step=1, unroll=2, carry=init)(
        lambda jv, c: step(jv, c))
    for k in range(QPW // LANES):
        carry = step(qblk + k, carry, diag_k=k)
    m1, m2, m3 = plsc.parallel_loop(
        qblk + QPW // LANES, N // LANES, step=1, unroll=2, carry=carry)(
        lambda jv, c: step(jv, c))

    third = np.float32(1.0 / 3.0)
    for u in range(Q32):
        h1 = plsc.unpack(m1[u], format=fmt)
        h2 = plsc.unpack(m2[u], format=fmt)
        h3 = plsc.unpack(m3[u], format=fmt)
        for half in range(2):
            mean = (_sqrt16(h1[half]) + _sqrt16(h2[half])
                    + _sqrt16(h3[half])) * third
            sc = jnp.maximum(mean, np.float32(1e-5)) * np.float32(0.001)
            dval = sc * sc
            sl = pl.ds(u * 32 + half * LANES, LANES)
            for k in range(9):
                outv[k, sl] = dval if k in (0, 4, 8) else zeros

    pltpu.sync_copy(outv, out_hbm.at[wid])


@jax.jit
def _knn(pts_t):
    mesh = plsc.VectorSubcoreMesh(
        core_axis_name="c", subcore_axis_name="s",
        num_cores=NC, num_subcores=NS)
    fn = functools.partial(
        pl.kernel,
        out_type=jax.ShapeDtypeStruct((NW, 9, QPW), jnp.float32),
        mesh=mesh,
        scratch_types=[
            pltpu.VMEM((3, N), jnp.float32),
            pltpu.VMEM((9, QPW), jnp.float32),
        ],
        compiler_params=pltpu.CompilerParams(needs_layout_passes=False),
    )(_knn_body)
    return fn(pts_t)


def kernel(points, colors):
    del colors
    pts_t = points.T
    out = _knn(pts_t)
    return jnp.transpose(out, (0, 2, 1)).reshape(N, 3, 3)
